# Initial kernel scaffold; baseline (speedup 1.0000x reference)
import jax
import jax.numpy as jnp
from jax.experimental import pallas as pl


def kernel(boxes, scores):
    raise NotImplementedError("write your pallas kernel here")


# speedup vs baseline: 20.0528x; 20.0528x over previous
import functools

import jax
import jax.numpy as jnp
import numpy as np
from jax import lax
from jax.experimental import pallas as pl
from jax.experimental.pallas import tpu as pltpu

_N = 20000
_ROWS = 160
_LANES = 128
_PAD = _ROWS * _LANES
_K = 1000
_SCORE_THRESH = 0.05
_NMS_THRESH = 0.5
_DETS = 300
_OUT_ROWS = 304

_NEG = -1.0
_BIGI = 2**30
_NEGF = -3.0e38


def _nms_kernel(x1_ref, y1_ref, x2_ref, y2_ref, s_ref, out_ref, sel_ref, a2_ref):
    shape = (_ROWS, _LANES)
    row_id = lax.broadcasted_iota(jnp.int32, shape, 0)
    lane_id = lax.broadcasted_iota(jnp.int32, shape, 1)
    idx = row_id * _LANES + lane_id
    valid = idx < _N

    x1 = x1_ref[:]
    y1 = y1_ref[:]
    x2 = x2_ref[:]
    y2 = y2_ref[:]

    probs = jax.nn.sigmoid(s_ref[:])
    probs = jnp.where(probs >= _SCORE_THRESH, probs, _NEG)
    probs = jnp.where(valid, probs, _NEG)

    bits = lax.bitcast_convert_type(probs, jnp.int32)
    keys = jnp.where(probs >= 0.0, bits, np.int32(-1))
    keys = jnp.where(valid, keys, np.int32(-2))

    def bs_body(_, lh):
        lo, hi = lh
        mid = lo + (hi - lo) // 2
        c = jnp.sum(jnp.where(keys > mid, 1, 0))
        take_hi = c < _K
        return (jnp.where(take_hi, lo, mid), jnp.where(take_hi, mid, hi))

    lo0 = np.int32(-2)
    hi0 = np.int32(2**30)
    _, t_key = lax.fori_loop(0, 31, bs_body, (lo0, hi0))

    c_gt = jnp.sum(jnp.where(keys > t_key, 1, 0))
    k_rem = _K - c_gt
    is_tie = keys == t_key

    def ts_body(_, lh):
        lo, hi = lh
        mid = lo + (hi - lo) // 2
        c = jnp.sum(jnp.where(is_tie & (idx < mid), 1, 0))
        take_hi = c >= k_rem
        return (jnp.where(take_hi, lo, mid), jnp.where(take_hi, mid, hi))

    _, m_hi = lax.fori_loop(0, 16, ts_body, (np.int32(0), np.int32(_PAD)))
    m_idx = jnp.where(k_rem > 0, m_hi, np.int32(0))

    cand = (keys > t_key) | (is_tie & (idx < m_idx))
    sel_ref[:] = jnp.where(cand, probs, _NEG)
    a2_ref[:] = (jnp.maximum(x2 - x1, 0.0) * jnp.maximum(y2 - y1, 0.0))

    def nms_body(t, _):
        sel = sel_ref[:]
        m = jnp.max(sel)
        j = jnp.min(jnp.where(sel == m, idx, np.int32(_BIGI)))
        is_j = idx == j
        bx1 = jnp.max(jnp.where(is_j, x1, _NEGF))
        by1 = jnp.max(jnp.where(is_j, y1, _NEGF))
        bx2 = jnp.max(jnp.where(is_j, x2, _NEGF))
        by2 = jnp.max(jnp.where(is_j, y2, _NEGF))
        keep = m > 0.0

        out_lane = lax.broadcasted_iota(jnp.int32, (1, _LANES), 1)
        vals = jnp.where(out_lane == 0, bx1, 0.0)
        vals = jnp.where(out_lane == 1, by1, vals)
        vals = jnp.where(out_lane == 2, bx2, vals)
        vals = jnp.where(out_lane == 3, by2, vals)
        vals = jnp.where(out_lane == 4, m, vals)
        vals = jnp.where(keep, vals, 0.0)
        out_ref[pl.ds(t, 1), :] = vals

        xx1 = jnp.maximum(bx1, x1)
        yy1 = jnp.maximum(by1, y1)
        xx2 = jnp.minimum(bx2, x2)
        yy2 = jnp.minimum(by2, y2)
        inter = jnp.maximum(xx2 - xx1, 0.0) * jnp.maximum(yy2 - yy1, 0.0)
        a1 = jnp.maximum(bx2 - bx1, 0.0) * jnp.maximum(by2 - by1, 0.0)
        union = jnp.maximum(a1 + a2_ref[:] - inter, 1e-8)
        iou = inter / union
        supp = (iou > _NMS_THRESH) | is_j
        sel_ref[:] = jnp.where(supp, _NEG, sel)
        return 0

    lax.fori_loop(0, _DETS, nms_body, 0)


@functools.partial(jax.jit, static_argnames=())
def kernel(boxes, scores):
    pad = _PAD - _N
    planes = []
    for c in range(4):
        p = jnp.pad(boxes[:, c], (0, pad)).reshape(_ROWS, _LANES)
        planes.append(p)
    s2d = jnp.pad(scores, (0, pad)).reshape(_ROWS, _LANES)

    out = pl.pallas_call(
        _nms_kernel,
        out_shape=jax.ShapeDtypeStruct((_OUT_ROWS, _LANES), jnp.float32),
        scratch_shapes=[
            pltpu.VMEM((_ROWS, _LANES), jnp.float32),
            pltpu.VMEM((_ROWS, _LANES), jnp.float32),
        ],
    )(*planes, s2d)

    out_boxes = out[:_DETS, :4]
    out_scores = out[:_DETS, 4]
    return out_boxes, out_scores
